# baseline (speedup 1.0000x reference)
import functools

import jax
import jax.numpy as jnp
from jax import lax
from jax.experimental import pallas as pl
from jax.experimental.pallas import tpu as pltpu
from jax.experimental.pallas import tpu_sc as plsc

_N = 64
_TAB = 32


@functools.cache
def _build_sigmoid_sc():
    mesh = plsc.ScalarSubcoreMesh(axis_name="c", num_cores=1)

    @functools.partial(
        pl.kernel,
        out_type=jax.ShapeDtypeStruct((_N,), jnp.float32),
        mesh=mesh,
        scratch_types=[
            pltpu.SMEM((_N,), jnp.float32),
            pltpu.SMEM((_N,), jnp.float32),
            pltpu.SMEM((_TAB,), jnp.float32),
        ],
    )
    def _sigmoid_sc(prob_hbm, out_hbm, xs, ys, tab):
        pltpu.sync_copy(prob_hbm, xs)

        def fill(i, v):
            tab[i] = v
            return v * 0.5

        lax.fori_loop(0, _TAB, fill, jnp.float32(1.0))

        log2e = jnp.float32(1.4426950408889634)
        ln2 = jnp.float32(0.6931471805599453)

        def step(i, carry):
            x = xs[i]
            ax = jnp.minimum(jnp.abs(x) * log2e, jnp.float32(30.0))
            n = (ax + 0.5).astype(jnp.int32)
            f = ax - n.astype(jnp.float32)
            y = -f * ln2
            p = 1.0 + y * (1.0 + y * (0.5 + y * (
                jnp.float32(1.0 / 6.0) + y * jnp.float32(1.0 / 24.0))))
            e = tab[n] * p
            d = 1.0 + e
            r = jnp.float32(1.45710678) - jnp.float32(0.5) * d
            r = r * (2.0 - d * r)
            r = r * (2.0 - d * r)
            r = r * (2.0 - d * r)
            ys[i] = jnp.where(x >= 0, r, e * r)
            return carry

        lax.fori_loop(0, _N, step, 0)
        pltpu.sync_copy(ys, out_hbm)

    return _sigmoid_sc


def kernel(prob):
    return _build_sigmoid_sc()(prob)


# speedup vs baseline: 1.0630x; 1.0630x over previous
import functools

import jax
import jax.numpy as jnp
from jax import lax
from jax.experimental import pallas as pl
from jax.experimental.pallas import tpu as pltpu
from jax.experimental.pallas import tpu_sc as plsc

_N = 64
_L = 16


@functools.cache
def _build_sigmoid_sc():
    mesh = plsc.VectorSubcoreMesh(
        core_axis_name="c", subcore_axis_name="s", num_cores=1, num_subcores=16
    )

    @functools.partial(
        pl.kernel,
        out_type=jax.ShapeDtypeStruct((_N,), jnp.float32),
        mesh=mesh,
        scratch_types=[pltpu.VMEM((_L,), jnp.float32)],
    )
    def _sigmoid_sc(prob_hbm, out_hbm, buf):
        sid = lax.axis_index("s")

        @pl.when(sid < _N // _L)
        def _():
            base = sid * _L
            pltpu.sync_copy(prob_hbm.at[pl.ds(base, _L)], buf)
            x = buf[...]
            buf[...] = 1.0 / (1.0 + jnp.exp(-x))
            pltpu.sync_copy(buf, out_hbm.at[pl.ds(base, _L)])

    return _sigmoid_sc


def kernel(prob):
    return _build_sigmoid_sc()(prob)
